# Initial kernel scaffold; baseline (speedup 1.0000x reference)
import jax
import jax.numpy as jnp
from jax.experimental import pallas as pl


def kernel(x, edge_index, edge_weight, W0_r, W1_r, b_r, W0_z, W1_z, b_z, W0_h, W1_h, b_h, prelu_w, W_lin, b_lin):
    raise NotImplementedError("write your pallas kernel here")


# speedup vs baseline: 8.6078x; 8.6078x over previous
import functools

import jax
import jax.numpy as jnp
from jax import lax
from jax.experimental import pallas as pl
from jax.experimental.pallas import tpu as pltpu
from jax.experimental.pallas import tpu_sc as plsc

N = 100000
E = 1600000
F_IN = 35
F_PAD = 48
L = 16
NCHUNK = 3
NC = 2
NS = 16
NW = NC * NS
EB = 400
EPT = E // NW
NBLK = EPT // EB
N_PAD = 100352
ROWS_PER_TILE = N_PAD // NS
ZROWS = 392


def _sc_segment_sum(xt0, xt1, xt2, src, dst, w):
    mesh = plsc.VectorSubcoreMesh(core_axis_name="c", subcore_axis_name="s")

    @functools.partial(
        pl.kernel,
        out_type=jax.ShapeDtypeStruct((NC * NCHUNK * N_PAD, L), jnp.float32),
        mesh=mesh,
        compiler_params=pltpu.CompilerParams(use_tc_tiling_on_sc=False),
        scratch_types=[
            pltpu.VMEM_SHARED((N_PAD, L), jnp.float32),
            pltpu.VMEM((EB,), jnp.int32),
            pltpu.VMEM((EB,), jnp.int32),
            pltpu.VMEM((EB,), jnp.float32),
            pltpu.VMEM((EB, L), jnp.float32),
            pltpu.VMEM((ZROWS, L), jnp.float32),
            pltpu.SemaphoreType.DMA,
        ],
    )
    def body(x0_hbm, x1_hbm, x2_hbm, src_hbm, dst_hbm, w_hbm, out_hbm,
             acc, srcv, dstv, wv, rows, zbuf, gsem):
        c = lax.axis_index("c")
        s = lax.axis_index("s")
        base_edge = (c * NS + s) * EPT
        my_row0 = s * ROWS_PER_TILE

        def fill_zero(i, _):
            zbuf[i, :] = jnp.zeros((L,), jnp.float32)
            return 0

        lax.fori_loop(0, ZROWS, fill_zero, 0)

        tables = (x0_hbm, x1_hbm, x2_hbm)
        for k in range(NCHUNK):
            def zero_slab(i, _):
                pltpu.sync_copy(zbuf, acc.at[pl.ds(my_row0 + i * ZROWS, ZROWS)])
                return 0

            lax.fori_loop(0, ROWS_PER_TILE // ZROWS, zero_slab, 0)
            plsc.subcore_barrier()

            def edge_block(b, _):
                off = base_edge + b * EB
                pltpu.sync_copy(src_hbm.at[pl.ds(off, EB)], srcv)
                pltpu.sync_copy(dst_hbm.at[pl.ds(off, EB)], dstv)
                pltpu.sync_copy(w_hbm.at[pl.ds(off, EB)], wv)
                pltpu.async_copy(tables[k].at[srcv], rows, gsem).wait()

                def scale(g, _):
                    w16 = wv[pl.ds(g * L, L)]
                    for j in range(L):
                        i = g * L + j
                        rows[i, :] = rows[i, :] * w16[j]
                    return 0

                lax.fori_loop(0, EB // L, scale, 0)
                pltpu.sync_copy(rows, acc.at[dstv], add=True)
                return 0

            lax.fori_loop(0, NBLK, edge_block, 0)
            plsc.subcore_barrier()

            out_row0 = (c * NCHUNK + k) * N_PAD + my_row0
            pltpu.sync_copy(acc.at[pl.ds(my_row0, ROWS_PER_TILE)],
                            out_hbm.at[pl.ds(out_row0, ROWS_PER_TILE)])
            plsc.subcore_barrier()

    return body(xt0, xt1, xt2, src, dst, w)


def _epilogue(xp, sp, wxz, wsz, wxh, wsh, bz, bh, pw, wlin, blin):
    R = 2000
    grid = (N // R,)

    def body(x_ref, sp_ref, wxz_ref, wsz_ref, wxh_ref, wsh_ref,
             bz_ref, bh_ref, pw_ref, wlin_ref, blin_ref,
             a0_ref, a1_ref, a2_ref, o_ref):
        s0 = sp_ref[0, 0] + sp_ref[1, 0]
        s1 = sp_ref[0, 1] + sp_ref[1, 1]
        s2 = sp_ref[0, 2] + sp_ref[1, 2]
        xb = x_ref[...]
        uz = (jnp.dot(xb, wxz_ref[...], preferred_element_type=jnp.float32)
              + jnp.dot(s0, wsz_ref[0:L, :], preferred_element_type=jnp.float32)
              + jnp.dot(s1, wsz_ref[L:2 * L, :], preferred_element_type=jnp.float32)
              + jnp.dot(s2, wsz_ref[2 * L:3 * L, :], preferred_element_type=jnp.float32)
              + bz_ref[...])
        uh = (jnp.dot(xb, wxh_ref[...], preferred_element_type=jnp.float32)
              + jnp.dot(s0, wsh_ref[0:L, :], preferred_element_type=jnp.float32)
              + jnp.dot(s1, wsh_ref[L:2 * L, :], preferred_element_type=jnp.float32)
              + jnp.dot(s2, wsh_ref[2 * L:3 * L, :], preferred_element_type=jnp.float32)
              + bh_ref[...])
        z = jax.nn.sigmoid(uz)
        ht = jnp.tanh(uh)
        hn = (1.0 - z) * ht
        p = jnp.maximum(hn, 0.0) + pw_ref[0, 0] * jnp.minimum(hn, 0.0)
        o_ref[...] = (jnp.sum(p * wlin_ref[...], axis=1, keepdims=True)
                      + blin_ref[0, 0])
        a0_ref[...] = s0
        a1_ref[...] = s1
        a2_ref[...] = s2

    full = lambda shape: pl.BlockSpec(shape, lambda i: (0,) * len(shape))
    return pl.pallas_call(
        body,
        grid=grid,
        in_specs=[
            pl.BlockSpec((R, F_PAD), lambda i: (i, 0)),
            pl.BlockSpec((NC, NCHUNK, R, L), lambda i: (0, 0, i, 0)),
            full((F_PAD, 3)), full((F_PAD, 3)),
            full((F_PAD, 3)), full((F_PAD, 3)),
            full((1, 3)), full((1, 3)), full((1, 1)),
            full((1, 3)), full((1, 1)),
        ],
        out_specs=[
            pl.BlockSpec((R, L), lambda i: (i, 0)),
            pl.BlockSpec((R, L), lambda i: (i, 0)),
            pl.BlockSpec((R, L), lambda i: (i, 0)),
            pl.BlockSpec((R, 1), lambda i: (i, 0)),
        ],
        out_shape=[
            jax.ShapeDtypeStruct((N, L), jnp.float32),
            jax.ShapeDtypeStruct((N, L), jnp.float32),
            jax.ShapeDtypeStruct((N, L), jnp.float32),
            jax.ShapeDtypeStruct((N, 1), jnp.float32),
        ],
    )(xp, sp, wxz, wsz, wxh, wsh, bz, bh, pw, wlin, blin)


def kernel(x, edge_index, edge_weight, W0_r, W1_r, b_r, W0_z, W1_z, b_z,
           W0_h, W1_h, b_h, prelu_w, W_lin, b_lin):
    xp = jnp.pad(x, ((0, 0), (0, F_PAD - F_IN)))
    xt = xp.reshape(N, NCHUNK, L).transpose(1, 0, 2)
    src = edge_index[0]
    dst = edge_index[1]

    sp_flat = _sc_segment_sum(xt[0], xt[1], xt[2], src, dst, edge_weight)
    sp = sp_flat.reshape(NC, NCHUNK, N_PAD, L)

    pad_w = lambda m: jnp.pad(m[:F_IN, :], ((0, F_PAD - F_IN), (0, 0)))
    wxz = pad_w(W0_z)
    wsz = pad_w(W1_z)
    wxh = pad_w(W0_h)
    wsh = pad_w(W1_h)
    bz = b_z.reshape(1, 3)
    bh = b_h.reshape(1, 3)
    pw = prelu_w.reshape(1, 1)
    wlin = W_lin.reshape(1, 3)
    blin = b_lin.reshape(1, 1)

    a0, a1, a2, out = _epilogue(xp, sp, wxz, wsz, wxh, wsh, bz, bh, pw,
                                wlin, blin)
    A = jnp.concatenate([a0, a1, a2[:, :F_IN + 3 - 2 * L]], axis=1)
    return (out, A, A, A)


# speedup vs baseline: 12.6360x; 1.4680x over previous
import functools

import jax
import jax.numpy as jnp
from jax import lax
from jax.experimental import pallas as pl
from jax.experimental.pallas import tpu as pltpu
from jax.experimental.pallas import tpu_sc as plsc

N = 100000
E = 1600000
F_IN = 35
F_PAD = 48
L = 16
NCHUNK = 3
NC = 2
NS = 16
NW = NC * NS
EB = 400
EPT = 50400
EP = NW * EPT
NBLK = EPT // EB
HALF = NBLK // 2
N_PAD = 100352
ROWS_PER_TILE = N_PAD // NS
ZROWS = 392


def _sc_segment_sum(xt0, xt1, xt2, src, dst, w):
    mesh = plsc.VectorSubcoreMesh(core_axis_name="c", subcore_axis_name="s")

    @functools.partial(
        pl.kernel,
        out_type=jax.ShapeDtypeStruct((NC * NCHUNK * N_PAD, L), jnp.float32),
        mesh=mesh,
        compiler_params=pltpu.CompilerParams(use_tc_tiling_on_sc=False),
        scratch_types=[
            pltpu.VMEM_SHARED((N_PAD, L), jnp.float32),
            pltpu.VMEM((EB,), jnp.int32),
            pltpu.VMEM((EB,), jnp.int32),
            pltpu.VMEM((EB,), jnp.float32),
            pltpu.VMEM((EB, L), jnp.float32),
            pltpu.VMEM((EB,), jnp.int32),
            pltpu.VMEM((EB,), jnp.int32),
            pltpu.VMEM((EB,), jnp.float32),
            pltpu.VMEM((EB, L), jnp.float32),
            pltpu.VMEM((ZROWS, L), jnp.float32),
            pltpu.SemaphoreType.DMA,
            pltpu.SemaphoreType.DMA,
            pltpu.SemaphoreType.DMA,
            pltpu.SemaphoreType.DMA,
            pltpu.SemaphoreType.DMA,
            pltpu.SemaphoreType.DMA,
        ],
    )
    def body(x0_hbm, x1_hbm, x2_hbm, src_hbm, dst_hbm, w_hbm, out_hbm,
             acc, srcv0, dstv0, wv0, rows0, srcv1, dstv1, wv1, rows1, zbuf,
             isem0, isem1, gsem0, gsem1, ssem0, ssem1):
        c = lax.axis_index("c")
        s = lax.axis_index("s")
        base_edge = (c * NS + s) * EPT
        my_row0 = s * ROWS_PER_TILE

        def fill_zero(i, _):
            zbuf[i, :] = jnp.zeros((L,), jnp.float32)
            return 0

        lax.fori_loop(0, ZROWS, fill_zero, 0)

        def issue_idx(off, srcv, dstv, wv, sem):
            pltpu.async_copy(src_hbm.at[pl.ds(off, EB)], srcv, sem)
            pltpu.async_copy(dst_hbm.at[pl.ds(off, EB)], dstv, sem)
            pltpu.async_copy(w_hbm.at[pl.ds(off, EB)], wv, sem)

        def drain_idx(srcv, dstv, wv, sem):
            pltpu.make_async_copy(src_hbm.at[pl.ds(0, EB)], srcv, sem).wait()
            pltpu.make_async_copy(dst_hbm.at[pl.ds(0, EB)], dstv, sem).wait()
            pltpu.make_async_copy(w_hbm.at[pl.ds(0, EB)], wv, sem).wait()

        def drain_rows(tab, rows, sem):
            pltpu.make_async_copy(tab.at[pl.ds(0, EB)], rows, sem).wait()

        def scale_rows(rows, wv):
            def scale(g, _):
                w16 = wv[pl.ds(g * L, L)]
                for j in range(L):
                    i = g * L + j
                    rows[i, :] = rows[i, :] * w16[j]
                return 0

            lax.fori_loop(0, EB // L, scale, 0)

        tables = (x0_hbm, x1_hbm, x2_hbm)
        for k in range(NCHUNK):
            tab = tables[k]

            def zero_slab(i, _):
                pltpu.sync_copy(zbuf, acc.at[pl.ds(my_row0 + i * ZROWS, ZROWS)])
                return 0

            lax.fori_loop(0, ROWS_PER_TILE // ZROWS, zero_slab, 0)
            plsc.subcore_barrier()

            issue_idx(base_edge, srcv0, dstv0, wv0, isem0)
            issue_idx(base_edge + EB, srcv1, dstv1, wv1, isem1)
            drain_idx(srcv0, dstv0, wv0, isem0)
            pltpu.async_copy(tab.at[srcv0], rows0, gsem0)

            def pair(b2, _):
                off = base_edge + 2 * b2 * EB

                drain_idx(srcv1, dstv1, wv1, isem1)

                @pl.when(b2 > 0)
                def _():
                    drain_rows(tab, rows1, ssem1)

                pltpu.async_copy(tab.at[srcv1], rows1, gsem1)

                drain_rows(tab, rows0, gsem0)
                scale_rows(rows0, wv0)
                pltpu.async_copy(rows0, acc.at[dstv0], ssem0, add=True)

                @pl.when(b2 < HALF - 1)
                def _():
                    issue_idx(off + 2 * EB, srcv0, dstv0, wv0, isem0)

                @pl.when(b2 < HALF - 1)
                def _():
                    drain_idx(srcv0, dstv0, wv0, isem0)
                    drain_rows(tab, rows0, ssem0)
                    pltpu.async_copy(tab.at[srcv0], rows0, gsem0)

                drain_rows(tab, rows1, gsem1)
                scale_rows(rows1, wv1)
                pltpu.async_copy(rows1, acc.at[dstv1], ssem1, add=True)

                @pl.when(b2 < HALF - 1)
                def _():
                    issue_idx(off + 3 * EB, srcv1, dstv1, wv1, isem1)

                return 0

            lax.fori_loop(0, HALF, pair, 0)
            drain_rows(tab, rows0, ssem0)
            drain_rows(tab, rows1, ssem1)
            plsc.subcore_barrier()

            out_row0 = (c * NCHUNK + k) * N_PAD + my_row0
            pltpu.sync_copy(acc.at[pl.ds(my_row0, ROWS_PER_TILE)],
                            out_hbm.at[pl.ds(out_row0, ROWS_PER_TILE)])
            plsc.subcore_barrier()

    return body(xt0, xt1, xt2, src, dst, w)


def _epilogue(xp, sp, wxz, wsz, wxh, wsh, bz, bh, pw, wlin, blin):
    R = 2000
    grid = (N // R,)

    def body(x_ref, sp_ref, wxz_ref, wsz_ref, wxh_ref, wsh_ref,
             bz_ref, bh_ref, pw_ref, wlin_ref, blin_ref,
             a0_ref, a1_ref, a2_ref, o_ref):
        s0 = sp_ref[0, 0] + sp_ref[1, 0]
        s1 = sp_ref[0, 1] + sp_ref[1, 1]
        s2 = sp_ref[0, 2] + sp_ref[1, 2]
        xb = x_ref[...]
        uz = (jnp.dot(xb, wxz_ref[...], preferred_element_type=jnp.float32)
              + jnp.dot(s0, wsz_ref[0:L, :], preferred_element_type=jnp.float32)
              + jnp.dot(s1, wsz_ref[L:2 * L, :], preferred_element_type=jnp.float32)
              + jnp.dot(s2, wsz_ref[2 * L:3 * L, :], preferred_element_type=jnp.float32)
              + bz_ref[...])
        uh = (jnp.dot(xb, wxh_ref[...], preferred_element_type=jnp.float32)
              + jnp.dot(s0, wsh_ref[0:L, :], preferred_element_type=jnp.float32)
              + jnp.dot(s1, wsh_ref[L:2 * L, :], preferred_element_type=jnp.float32)
              + jnp.dot(s2, wsh_ref[2 * L:3 * L, :], preferred_element_type=jnp.float32)
              + bh_ref[...])
        z = jax.nn.sigmoid(uz)
        ht = jnp.tanh(uh)
        hn = (1.0 - z) * ht
        p = jnp.maximum(hn, 0.0) + pw_ref[0, 0] * jnp.minimum(hn, 0.0)
        o_ref[...] = (jnp.sum(p * wlin_ref[...], axis=1, keepdims=True)
                      + blin_ref[0, 0])
        a0_ref[...] = s0
        a1_ref[...] = s1
        a2_ref[...] = s2

    full = lambda shape: pl.BlockSpec(shape, lambda i: (0,) * len(shape))
    return pl.pallas_call(
        body,
        grid=grid,
        in_specs=[
            pl.BlockSpec((R, F_PAD), lambda i: (i, 0)),
            pl.BlockSpec((NC, NCHUNK, R, L), lambda i: (0, 0, i, 0)),
            full((F_PAD, 3)), full((F_PAD, 3)),
            full((F_PAD, 3)), full((F_PAD, 3)),
            full((1, 3)), full((1, 3)), full((1, 1)),
            full((1, 3)), full((1, 1)),
        ],
        out_specs=[
            pl.BlockSpec((R, L), lambda i: (i, 0)),
            pl.BlockSpec((R, L), lambda i: (i, 0)),
            pl.BlockSpec((R, L), lambda i: (i, 0)),
            pl.BlockSpec((R, 1), lambda i: (i, 0)),
        ],
        out_shape=[
            jax.ShapeDtypeStruct((N, L), jnp.float32),
            jax.ShapeDtypeStruct((N, L), jnp.float32),
            jax.ShapeDtypeStruct((N, L), jnp.float32),
            jax.ShapeDtypeStruct((N, 1), jnp.float32),
        ],
    )(xp, sp, wxz, wsz, wxh, wsh, bz, bh, pw, wlin, blin)


def kernel(x, edge_index, edge_weight, W0_r, W1_r, b_r, W0_z, W1_z, b_z,
           W0_h, W1_h, b_h, prelu_w, W_lin, b_lin):
    xp = jnp.pad(x, ((0, 0), (0, F_PAD - F_IN)))
    xt = xp.reshape(N, NCHUNK, L).transpose(1, 0, 2)
    pad_e = EP - E
    src = jnp.concatenate([edge_index[0], jnp.zeros((pad_e,), jnp.int32)])
    dst = jnp.concatenate([edge_index[1], jnp.zeros((pad_e,), jnp.int32)])
    w = jnp.concatenate([edge_weight, jnp.zeros((pad_e,), jnp.float32)])

    sp_flat = _sc_segment_sum(xt[0], xt[1], xt[2], src, dst, w)
    sp = sp_flat.reshape(NC, NCHUNK, N_PAD, L)

    pad_w = lambda m: jnp.pad(m[:F_IN, :], ((0, F_PAD - F_IN), (0, 0)))
    wxz = pad_w(W0_z)
    wsz = pad_w(W1_z)
    wxh = pad_w(W0_h)
    wsh = pad_w(W1_h)
    bz = b_z.reshape(1, 3)
    bh = b_h.reshape(1, 3)
    pw = prelu_w.reshape(1, 1)
    wlin = W_lin.reshape(1, 3)
    blin = b_lin.reshape(1, 1)

    a0, a1, a2, out = _epilogue(xp, sp, wxz, wsz, wxh, wsh, bz, bh, pw,
                                wlin, blin)
    A = jnp.concatenate([a0, a1, a2[:, :F_IN + 3 - 2 * L]], axis=1)
    return (out, A, A, A)


# speedup vs baseline: 14.0262x; 1.1100x over previous
import functools

import jax
import jax.numpy as jnp
from jax import lax
from jax.experimental import pallas as pl
from jax.experimental.pallas import tpu as pltpu
from jax.experimental.pallas import tpu_sc as plsc

N = 100000
E = 1600000
F_IN = 35
F_PAD = 48
L = 16
NCHUNK = 3
NC = 2
NS = 16
NW = NC * NS
EB = 400
EPT = E // NW
NBLK = EPT // EB
NPAIR = NBLK // 2
N_PAD = 100352
ROWS_PER_TILE = N_PAD // NS
ZROWS = 392


def _sc_segment_sum(xt0, xt1, xt2, src, dst, w):
    mesh = plsc.VectorSubcoreMesh(core_axis_name="c", subcore_axis_name="s")

    @functools.partial(
        pl.kernel,
        out_type=jax.ShapeDtypeStruct((NC * NCHUNK * N_PAD, L), jnp.float32),
        mesh=mesh,
        compiler_params=pltpu.CompilerParams(use_tc_tiling_on_sc=False),
        scratch_types=[
            pltpu.VMEM_SHARED((N_PAD, L), jnp.float32),
            pltpu.VMEM((EB,), jnp.int32),
            pltpu.VMEM((EB,), jnp.int32),
            pltpu.VMEM((EB,), jnp.float32),
            pltpu.VMEM((EB, L), jnp.float32),
            pltpu.VMEM((EB,), jnp.int32),
            pltpu.VMEM((EB,), jnp.int32),
            pltpu.VMEM((EB,), jnp.float32),
            pltpu.VMEM((EB, L), jnp.float32),
            pltpu.VMEM((ZROWS, L), jnp.float32),
            pltpu.SemaphoreType.DMA,
            pltpu.SemaphoreType.DMA,
            pltpu.SemaphoreType.DMA,
            pltpu.SemaphoreType.DMA,
            pltpu.SemaphoreType.DMA,
            pltpu.SemaphoreType.DMA,
        ],
    )
    def body(x0_hbm, x1_hbm, x2_hbm, src_hbm, dst_hbm, w_hbm, out_hbm,
             acc, srcv0, dstv0, wv0, rows0, srcv1, dstv1, wv1, rows1, zbuf,
             isem0, isem1, gsem0, gsem1, ssem0, ssem1):
        c = lax.axis_index("c")
        s = lax.axis_index("s")
        base_edge = (c * NS + s) * EPT
        my_row0 = s * ROWS_PER_TILE

        def fill_zero(i, _):
            zbuf[i, :] = jnp.zeros((L,), jnp.float32)
            return 0

        lax.fori_loop(0, ZROWS, fill_zero, 0)

        def issue_idx(off, srcv, dstv, wv, sem):
            pltpu.async_copy(src_hbm.at[pl.ds(off, EB)], srcv, sem)
            pltpu.async_copy(dst_hbm.at[pl.ds(off, EB)], dstv, sem)
            pltpu.async_copy(w_hbm.at[pl.ds(off, EB)], wv, sem)

        def drain_idx(srcv, dstv, wv, sem):
            pltpu.make_async_copy(src_hbm.at[pl.ds(0, EB)], srcv, sem).wait()
            pltpu.make_async_copy(dst_hbm.at[pl.ds(0, EB)], dstv, sem).wait()
            pltpu.make_async_copy(w_hbm.at[pl.ds(0, EB)], wv, sem).wait()

        def drain_rows(tab, rows, sem):
            pltpu.make_async_copy(tab.at[pl.ds(0, EB)], rows, sem).wait()

        def scale_rows(rows, wv):
            def scale(g, _):
                w16 = wv[pl.ds(g * L, L)]
                for j in range(L):
                    i = g * L + j
                    rows[i, :] = rows[i, :] * w16[j]
                return 0

            lax.fori_loop(0, EB // L, scale, 0)

        tables = (x0_hbm, x1_hbm, x2_hbm)
        for k in range(NCHUNK):
            tab = tables[k]

            def zero_slab(i, _):
                pltpu.sync_copy(zbuf, acc.at[pl.ds(my_row0 + i * ZROWS, ZROWS)])
                return 0

            lax.fori_loop(0, ROWS_PER_TILE // ZROWS, zero_slab, 0)
            plsc.subcore_barrier()

            issue_idx(base_edge, srcv0, dstv0, wv0, isem0)
            issue_idx(base_edge + EB, srcv1, dstv1, wv1, isem1)
            drain_idx(srcv0, dstv0, wv0, isem0)
            pltpu.async_copy(tab.at[srcv0], rows0, gsem0)

            def pair(b2, _):
                off = base_edge + 2 * b2 * EB

                drain_idx(srcv1, dstv1, wv1, isem1)

                @pl.when(b2 > 0)
                def _():
                    drain_rows(tab, rows1, ssem1)

                pltpu.async_copy(tab.at[srcv1], rows1, gsem1)

                drain_rows(tab, rows0, gsem0)
                scale_rows(rows0, wv0)
                pltpu.async_copy(rows0, acc.at[dstv0], ssem0, add=True)

                @pl.when(2 * b2 + 2 < NBLK)
                def _():
                    issue_idx(off + 2 * EB, srcv0, dstv0, wv0, isem0)

                @pl.when(2 * b2 + 2 < NBLK)
                def _():
                    drain_idx(srcv0, dstv0, wv0, isem0)
                    drain_rows(tab, rows0, ssem0)
                    pltpu.async_copy(tab.at[srcv0], rows0, gsem0)

                drain_rows(tab, rows1, gsem1)
                scale_rows(rows1, wv1)
                pltpu.async_copy(rows1, acc.at[dstv1], ssem1, add=True)

                @pl.when(2 * b2 + 3 < NBLK)
                def _():
                    issue_idx(off + 3 * EB, srcv1, dstv1, wv1, isem1)

                return 0

            lax.fori_loop(0, NPAIR, pair, 0)
            drain_rows(tab, rows0, gsem0)
            scale_rows(rows0, wv0)
            pltpu.async_copy(rows0, acc.at[dstv0], ssem0, add=True)
            drain_rows(tab, rows0, ssem0)
            drain_rows(tab, rows1, ssem1)
            plsc.subcore_barrier()

            out_row0 = (c * NCHUNK + k) * N_PAD + my_row0
            pltpu.sync_copy(acc.at[pl.ds(my_row0, ROWS_PER_TILE)],
                            out_hbm.at[pl.ds(out_row0, ROWS_PER_TILE)])
            plsc.subcore_barrier()

    return body(xt0, xt1, xt2, src, dst, w)


def _epilogue(xp, sp, wxz, wsz, wxh, wsh, bz, bh, pw, wlin, blin):
    R = 2000
    grid = (N // R,)

    def body(x_ref, sp_ref, wxz_ref, wsz_ref, wxh_ref, wsh_ref,
             bz_ref, bh_ref, pw_ref, wlin_ref, blin_ref,
             a0_ref, a1_ref, a2_ref, o_ref):
        s0 = sp_ref[0, 0] + sp_ref[1, 0]
        s1 = sp_ref[0, 1] + sp_ref[1, 1]
        s2 = sp_ref[0, 2] + sp_ref[1, 2]
        xb = x_ref[...]
        uz = (jnp.dot(xb, wxz_ref[...], preferred_element_type=jnp.float32)
              + jnp.dot(s0, wsz_ref[0:L, :], preferred_element_type=jnp.float32)
              + jnp.dot(s1, wsz_ref[L:2 * L, :], preferred_element_type=jnp.float32)
              + jnp.dot(s2, wsz_ref[2 * L:3 * L, :], preferred_element_type=jnp.float32)
              + bz_ref[...])
        uh = (jnp.dot(xb, wxh_ref[...], preferred_element_type=jnp.float32)
              + jnp.dot(s0, wsh_ref[0:L, :], preferred_element_type=jnp.float32)
              + jnp.dot(s1, wsh_ref[L:2 * L, :], preferred_element_type=jnp.float32)
              + jnp.dot(s2, wsh_ref[2 * L:3 * L, :], preferred_element_type=jnp.float32)
              + bh_ref[...])
        z = jax.nn.sigmoid(uz)
        ht = jnp.tanh(uh)
        hn = (1.0 - z) * ht
        p = jnp.maximum(hn, 0.0) + pw_ref[0, 0] * jnp.minimum(hn, 0.0)
        o_ref[...] = (jnp.sum(p * wlin_ref[...], axis=1, keepdims=True)
                      + blin_ref[0, 0])
        a0_ref[...] = s0
        a1_ref[...] = s1
        a2_ref[...] = s2

    full = lambda shape: pl.BlockSpec(shape, lambda i: (0,) * len(shape))
    return pl.pallas_call(
        body,
        grid=grid,
        in_specs=[
            pl.BlockSpec((R, F_PAD), lambda i: (i, 0)),
            pl.BlockSpec((NC, NCHUNK, R, L), lambda i: (0, 0, i, 0)),
            full((F_PAD, 3)), full((F_PAD, 3)),
            full((F_PAD, 3)), full((F_PAD, 3)),
            full((1, 3)), full((1, 3)), full((1, 1)),
            full((1, 3)), full((1, 1)),
        ],
        out_specs=[
            pl.BlockSpec((R, L), lambda i: (i, 0)),
            pl.BlockSpec((R, L), lambda i: (i, 0)),
            pl.BlockSpec((R, L), lambda i: (i, 0)),
            pl.BlockSpec((R, 1), lambda i: (i, 0)),
        ],
        out_shape=[
            jax.ShapeDtypeStruct((N, L), jnp.float32),
            jax.ShapeDtypeStruct((N, L), jnp.float32),
            jax.ShapeDtypeStruct((N, L), jnp.float32),
            jax.ShapeDtypeStruct((N, 1), jnp.float32),
        ],
    )(xp, sp, wxz, wsz, wxh, wsh, bz, bh, pw, wlin, blin)


def kernel(x, edge_index, edge_weight, W0_r, W1_r, b_r, W0_z, W1_z, b_z,
           W0_h, W1_h, b_h, prelu_w, W_lin, b_lin):
    xp = jnp.pad(x, ((0, 0), (0, F_PAD - F_IN)))
    xt = xp.reshape(N, NCHUNK, L).transpose(1, 0, 2)
    sp_flat = _sc_segment_sum(xt[0], xt[1], xt[2], edge_index[0],
                              edge_index[1], edge_weight)
    sp = sp_flat.reshape(NC, NCHUNK, N_PAD, L)

    pad_w = lambda m: jnp.pad(m[:F_IN, :], ((0, F_PAD - F_IN), (0, 0)))
    wxz = pad_w(W0_z)
    wsz = pad_w(W1_z)
    wxh = pad_w(W0_h)
    wsh = pad_w(W1_h)
    bz = b_z.reshape(1, 3)
    bh = b_h.reshape(1, 3)
    pw = prelu_w.reshape(1, 1)
    wlin = W_lin.reshape(1, 3)
    blin = b_lin.reshape(1, 1)

    a0, a1, a2, out = _epilogue(xp, sp, wxz, wsz, wxh, wsh, bz, bh, pw,
                                wlin, blin)
    A = jnp.concatenate([a0, a1, a2[:, :F_IN + 3 - 2 * L]], axis=1)
    return (out, A, A, A)


# speedup vs baseline: 14.8029x; 1.0554x over previous
import functools

import jax
import jax.numpy as jnp
from jax import lax
from jax.experimental import pallas as pl
from jax.experimental.pallas import tpu as pltpu
from jax.experimental.pallas import tpu_sc as plsc

N = 100000
E = 1600000
F_IN = 35
F_PAD = 48
L = 16
NCHUNK = 3
NC = 2
NS = 16
NW = NC * NS
EB = 400
EPT = E // NW
NBLK = EPT // EB
NPAIR = NBLK // 2
N_PAD = 100352
ROWS_PER_TILE = N_PAD // NS
ZROWS = 392


FMT_R = 1875


def _sc_format_table(xflat):
    mesh = plsc.VectorSubcoreMesh(core_axis_name="c", subcore_axis_name="s")
    rows_per_w = NCHUNK * N // NW

    @functools.partial(
        pl.kernel,
        out_type=jax.ShapeDtypeStruct((NCHUNK * N, L), jnp.float32),
        mesh=mesh,
        compiler_params=pltpu.CompilerParams(use_tc_tiling_on_sc=False),
        scratch_types=[
            pltpu.VMEM((FMT_R * L,), jnp.float32),
            pltpu.VMEM((FMT_R * L,), jnp.float32),
            pltpu.VMEM((FMT_R, L), jnp.float32),
            pltpu.SemaphoreType.DMA,
            pltpu.SemaphoreType.DMA,
        ],
    )
    def body(x_hbm, tab_hbm, buf0, buf1, buf2d, sem0, sem1):
        wid = lax.axis_index("c") * NS + lax.axis_index("s")
        row0 = wid * rows_per_w
        bufs = (buf0, buf1)
        sems = (sem0, sem1)
        n_it = rows_per_w // FMT_R

        def issue(i):
            r = row0 + i * FMT_R
            pltpu.async_copy(x_hbm.at[pl.ds(r * L, FMT_R * L)],
                             bufs[i % 2], sems[i % 2])

        issue(0)
        issue(1)
        for i in range(n_it):
            pltpu.make_async_copy(x_hbm.at[pl.ds(0, FMT_R * L)],
                                  bufs[i % 2], sems[i % 2]).wait()
            b1 = bufs[i % 2]

            def repack(r, _):
                buf2d[r, :] = b1[pl.ds(r * L, L)]
                return 0

            lax.fori_loop(0, FMT_R, repack, 0)
            pltpu.sync_copy(buf2d,
                            tab_hbm.at[pl.ds(row0 + i * FMT_R, FMT_R)])
            if i + 2 < n_it:
                issue(i + 2)

    return body(xflat)


def _sc_segment_sum(xtab, src, dst, w):
    mesh = plsc.VectorSubcoreMesh(core_axis_name="c", subcore_axis_name="s")

    @functools.partial(
        pl.kernel,
        out_type=jax.ShapeDtypeStruct((NC * NCHUNK * N_PAD, L), jnp.float32),
        mesh=mesh,
        compiler_params=pltpu.CompilerParams(use_tc_tiling_on_sc=False),
        scratch_types=[
            pltpu.VMEM_SHARED((N_PAD, L), jnp.float32),
            pltpu.VMEM((EB,), jnp.int32),
            pltpu.VMEM((EB,), jnp.int32),
            pltpu.VMEM((EB,), jnp.float32),
            pltpu.VMEM((EB, L), jnp.float32),
            pltpu.VMEM((EB,), jnp.int32),
            pltpu.VMEM((EB,), jnp.int32),
            pltpu.VMEM((EB,), jnp.float32),
            pltpu.VMEM((EB, L), jnp.float32),
            pltpu.VMEM((ZROWS, L), jnp.float32),
            pltpu.SemaphoreType.DMA,
            pltpu.SemaphoreType.DMA,
            pltpu.SemaphoreType.DMA,
            pltpu.SemaphoreType.DMA,
            pltpu.SemaphoreType.DMA,
            pltpu.SemaphoreType.DMA,
        ],
    )
    def body(xtab, src_hbm, dst_hbm, w_hbm, out_hbm,
             acc, srcv0, dstv0, wv0, rows0, srcv1, dstv1, wv1, rows1, zbuf,
             isem0, isem1, gsem0, gsem1, ssem0, ssem1):
        c = lax.axis_index("c")
        s = lax.axis_index("s")
        base_edge = (c * NS + s) * EPT
        my_row0 = s * ROWS_PER_TILE

        def fill_zero(i, _):
            zbuf[i, :] = jnp.zeros((L,), jnp.float32)
            return 0

        lax.fori_loop(0, ZROWS, fill_zero, 0)

        def issue_idx(off, srcv, dstv, wv, sem):
            pltpu.async_copy(src_hbm.at[pl.ds(off, EB)], srcv, sem)
            pltpu.async_copy(dst_hbm.at[pl.ds(off, EB)], dstv, sem)
            pltpu.async_copy(w_hbm.at[pl.ds(off, EB)], wv, sem)

        def drain_idx(srcv, dstv, wv, sem):
            pltpu.make_async_copy(src_hbm.at[pl.ds(0, EB)], srcv, sem).wait()
            pltpu.make_async_copy(dst_hbm.at[pl.ds(0, EB)], dstv, sem).wait()
            pltpu.make_async_copy(w_hbm.at[pl.ds(0, EB)], wv, sem).wait()

        def drain_rows(tab, rows, sem):
            pltpu.make_async_copy(tab.at[pl.ds(0, EB)], rows, sem).wait()

        def scale_rows(rows, wv):
            def scale(g, _):
                w16 = wv[pl.ds(g * L, L)]
                for j in range(L):
                    i = g * L + j
                    rows[i, :] = rows[i, :] * w16[j]
                return 0

            lax.fori_loop(0, EB // L, scale, 0)

        for k in range(NCHUNK):
            tab = xtab.at[pl.ds(k * N, N)]

            def zero_slab(i, _):
                pltpu.sync_copy(zbuf, acc.at[pl.ds(my_row0 + i * ZROWS, ZROWS)])
                return 0

            lax.fori_loop(0, ROWS_PER_TILE // ZROWS, zero_slab, 0)
            plsc.subcore_barrier()

            issue_idx(base_edge, srcv0, dstv0, wv0, isem0)
            issue_idx(base_edge + EB, srcv1, dstv1, wv1, isem1)
            drain_idx(srcv0, dstv0, wv0, isem0)
            pltpu.async_copy(tab.at[srcv0], rows0, gsem0)

            def pair(b2, _):
                off = base_edge + 2 * b2 * EB

                drain_idx(srcv1, dstv1, wv1, isem1)

                @pl.when(b2 > 0)
                def _():
                    drain_rows(tab, rows1, ssem1)

                pltpu.async_copy(tab.at[srcv1], rows1, gsem1)

                drain_rows(tab, rows0, gsem0)
                scale_rows(rows0, wv0)
                pltpu.async_copy(rows0, acc.at[dstv0], ssem0, add=True)

                @pl.when(2 * b2 + 2 < NBLK)
                def _():
                    issue_idx(off + 2 * EB, srcv0, dstv0, wv0, isem0)

                @pl.when(2 * b2 + 2 < NBLK)
                def _():
                    drain_idx(srcv0, dstv0, wv0, isem0)
                    drain_rows(tab, rows0, ssem0)
                    pltpu.async_copy(tab.at[srcv0], rows0, gsem0)

                drain_rows(tab, rows1, gsem1)
                scale_rows(rows1, wv1)
                pltpu.async_copy(rows1, acc.at[dstv1], ssem1, add=True)

                @pl.when(2 * b2 + 3 < NBLK)
                def _():
                    issue_idx(off + 3 * EB, srcv1, dstv1, wv1, isem1)

                return 0

            lax.fori_loop(0, NPAIR, pair, 0)
            drain_rows(tab, rows0, gsem0)
            scale_rows(rows0, wv0)
            pltpu.async_copy(rows0, acc.at[dstv0], ssem0, add=True)
            drain_rows(tab, rows0, ssem0)
            drain_rows(tab, rows1, ssem1)
            plsc.subcore_barrier()

            out_row0 = (c * NCHUNK + k) * N_PAD + my_row0
            pltpu.sync_copy(acc.at[pl.ds(my_row0, ROWS_PER_TILE)],
                            out_hbm.at[pl.ds(out_row0, ROWS_PER_TILE)])
            plsc.subcore_barrier()

    return body(xtab, src, dst, w)


def _epilogue(xp, sp, wxz, wsz, wxh, wsh, bz, bh, pw, wlin, blin):
    R = 2000
    grid = (N // R,)

    def body(x_ref, sp_ref, wxz_ref, wsz_ref, wxh_ref, wsh_ref,
             bz_ref, bh_ref, pw_ref, wlin_ref, blin_ref,
             a0_ref, a1_ref, a2_ref, o_ref):
        s0 = sp_ref[0, 0] + sp_ref[1, 0]
        s1 = sp_ref[0, 1] + sp_ref[1, 1]
        s2 = sp_ref[0, 2] + sp_ref[1, 2]
        xb = x_ref[...]
        uz = (jnp.dot(xb, wxz_ref[...], preferred_element_type=jnp.float32)
              + jnp.dot(s0, wsz_ref[0:L, :], preferred_element_type=jnp.float32)
              + jnp.dot(s1, wsz_ref[L:2 * L, :], preferred_element_type=jnp.float32)
              + jnp.dot(s2, wsz_ref[2 * L:3 * L, :], preferred_element_type=jnp.float32)
              + bz_ref[...])
        uh = (jnp.dot(xb, wxh_ref[...], preferred_element_type=jnp.float32)
              + jnp.dot(s0, wsh_ref[0:L, :], preferred_element_type=jnp.float32)
              + jnp.dot(s1, wsh_ref[L:2 * L, :], preferred_element_type=jnp.float32)
              + jnp.dot(s2, wsh_ref[2 * L:3 * L, :], preferred_element_type=jnp.float32)
              + bh_ref[...])
        z = jax.nn.sigmoid(uz)
        ht = jnp.tanh(uh)
        hn = (1.0 - z) * ht
        p = jnp.maximum(hn, 0.0) + pw_ref[0, 0] * jnp.minimum(hn, 0.0)
        o_ref[...] = (jnp.sum(p * wlin_ref[...], axis=1, keepdims=True)
                      + blin_ref[0, 0])
        a0_ref[...] = s0
        a1_ref[...] = s1
        a2_ref[...] = s2

    full = lambda shape: pl.BlockSpec(shape, lambda i: (0,) * len(shape))
    return pl.pallas_call(
        body,
        grid=grid,
        in_specs=[
            pl.BlockSpec((R, F_PAD), lambda i: (i, 0)),
            pl.BlockSpec((NC, NCHUNK, R, L), lambda i: (0, 0, i, 0)),
            full((F_PAD, 3)), full((F_PAD, 3)),
            full((F_PAD, 3)), full((F_PAD, 3)),
            full((1, 3)), full((1, 3)), full((1, 1)),
            full((1, 3)), full((1, 1)),
        ],
        out_specs=[
            pl.BlockSpec((R, L), lambda i: (i, 0)),
            pl.BlockSpec((R, L), lambda i: (i, 0)),
            pl.BlockSpec((R, L), lambda i: (i, 0)),
            pl.BlockSpec((R, 1), lambda i: (i, 0)),
        ],
        out_shape=[
            jax.ShapeDtypeStruct((N, L), jnp.float32),
            jax.ShapeDtypeStruct((N, L), jnp.float32),
            jax.ShapeDtypeStruct((N, L), jnp.float32),
            jax.ShapeDtypeStruct((N, 1), jnp.float32),
        ],
    )(xp, sp, wxz, wsz, wxh, wsh, bz, bh, pw, wlin, blin)


def kernel(x, edge_index, edge_weight, W0_r, W1_r, b_r, W0_z, W1_z, b_z,
           W0_h, W1_h, b_h, prelu_w, W_lin, b_lin):
    xp = jnp.pad(x, ((0, 0), (0, F_PAD - F_IN)))
    xflat = xp.reshape(N, NCHUNK, L).transpose(1, 0, 2).reshape(-1)
    xtab = _sc_format_table(xflat)
    sp_flat = _sc_segment_sum(xtab, edge_index[0], edge_index[1],
                              edge_weight)
    sp = sp_flat.reshape(NC, NCHUNK, N_PAD, L)

    pad_w = lambda m: jnp.pad(m[:F_IN, :], ((0, F_PAD - F_IN), (0, 0)))
    wxz = pad_w(W0_z)
    wsz = pad_w(W1_z)
    wxh = pad_w(W0_h)
    wsh = pad_w(W1_h)
    bz = b_z.reshape(1, 3)
    bh = b_h.reshape(1, 3)
    pw = prelu_w.reshape(1, 1)
    wlin = W_lin.reshape(1, 3)
    blin = b_lin.reshape(1, 1)

    a0, a1, a2, out = _epilogue(xp, sp, wxz, wsz, wxh, wsh, bz, bh, pw,
                                wlin, blin)
    A = jnp.concatenate([a0, a1, a2[:, :F_IN + 3 - 2 * L]], axis=1)
    return (out, A, A, A)


# speedup vs baseline: 14.8673x; 1.0044x over previous
import functools

import jax
import jax.numpy as jnp
from jax import lax
from jax.experimental import pallas as pl
from jax.experimental.pallas import tpu as pltpu
from jax.experimental.pallas import tpu_sc as plsc

N = 100000
E = 1600000
F_IN = 35
F_PAD = 48
L = 16
NCHUNK = 3
NC = 2
NS = 16
NW = NC * NS
EB = 400
EPT = E // NW
NBLK = EPT // EB
NPAIR = NBLK // 2
N_PAD = 100352
ROWS_PER_TILE = N_PAD // NS
ZROWS = 392


FMT_R = 1875


def _sc_format_table(xflat):
    mesh = plsc.VectorSubcoreMesh(core_axis_name="c", subcore_axis_name="s")
    rows_per_w = NCHUNK * N // NW

    @functools.partial(
        pl.kernel,
        out_type=jax.ShapeDtypeStruct((NCHUNK * N, L), jnp.float32),
        mesh=mesh,
        compiler_params=pltpu.CompilerParams(use_tc_tiling_on_sc=False),
        scratch_types=[
            pltpu.VMEM((FMT_R * L,), jnp.float32),
            pltpu.VMEM((FMT_R * L,), jnp.float32),
            pltpu.VMEM((FMT_R, L), jnp.float32),
            pltpu.SemaphoreType.DMA,
            pltpu.SemaphoreType.DMA,
        ],
    )
    def body(x_hbm, tab_hbm, buf0, buf1, buf2d, sem0, sem1):
        wid = lax.axis_index("c") * NS + lax.axis_index("s")
        row0 = wid * rows_per_w
        bufs = (buf0, buf1)
        sems = (sem0, sem1)
        n_it = rows_per_w // FMT_R

        def issue(i):
            r = row0 + i * FMT_R
            pltpu.async_copy(x_hbm.at[pl.ds(r * L, FMT_R * L)],
                             bufs[i % 2], sems[i % 2])

        issue(0)
        issue(1)
        for i in range(n_it):
            pltpu.make_async_copy(x_hbm.at[pl.ds(0, FMT_R * L)],
                                  bufs[i % 2], sems[i % 2]).wait()
            b1 = bufs[i % 2]

            def repack(r, _):
                buf2d[r, :] = b1[pl.ds(r * L, L)]
                return 0

            lax.fori_loop(0, FMT_R, repack, 0)
            pltpu.sync_copy(buf2d,
                            tab_hbm.at[pl.ds(row0 + i * FMT_R, FMT_R)])
            if i + 2 < n_it:
                issue(i + 2)

    return body(xflat)


def _sc_segment_sum(xtab, src, dst, w):
    mesh = plsc.VectorSubcoreMesh(core_axis_name="c", subcore_axis_name="s")

    @functools.partial(
        pl.kernel,
        out_type=jax.ShapeDtypeStruct((NC * NCHUNK * N_PAD, L), jnp.float32),
        mesh=mesh,
        compiler_params=pltpu.CompilerParams(use_tc_tiling_on_sc=False),
        scratch_types=[
            pltpu.VMEM_SHARED((N_PAD, L), jnp.float32),
            pltpu.VMEM((EB,), jnp.int32),
            pltpu.VMEM((EB,), jnp.int32),
            pltpu.VMEM((EB,), jnp.float32),
            pltpu.VMEM((EB, L), jnp.float32),
            pltpu.VMEM((EB,), jnp.int32),
            pltpu.VMEM((EB,), jnp.int32),
            pltpu.VMEM((EB,), jnp.float32),
            pltpu.VMEM((EB, L), jnp.float32),
            pltpu.VMEM((ZROWS, L), jnp.float32),
            pltpu.SemaphoreType.DMA,
            pltpu.SemaphoreType.DMA,
            pltpu.SemaphoreType.DMA,
            pltpu.SemaphoreType.DMA,
            pltpu.SemaphoreType.DMA,
            pltpu.SemaphoreType.DMA,
        ],
    )
    def body(xtab, src_hbm, dst_hbm, w_hbm, out_hbm,
             acc, srcv0, dstv0, wv0, rows0, srcv1, dstv1, wv1, rows1, zbuf,
             isem0, isem1, gsem0, gsem1, ssem0, ssem1):
        c = lax.axis_index("c")
        s = lax.axis_index("s")
        base_edge = (c * NS + s) * EPT
        my_row0 = s * ROWS_PER_TILE

        def fill_zero(i, _):
            zbuf[i, :] = jnp.zeros((L,), jnp.float32)
            return 0

        lax.fori_loop(0, ZROWS, fill_zero, 0)

        def issue_idx(off, srcv, dstv, wv, sem):
            pltpu.async_copy(src_hbm.at[pl.ds(off, EB)], srcv, sem)
            pltpu.async_copy(dst_hbm.at[pl.ds(off, EB)], dstv, sem)
            pltpu.async_copy(w_hbm.at[pl.ds(off, EB)], wv, sem)

        def drain_idx(srcv, dstv, wv, sem):
            pltpu.make_async_copy(src_hbm.at[pl.ds(0, EB)], srcv, sem).wait()
            pltpu.make_async_copy(dst_hbm.at[pl.ds(0, EB)], dstv, sem).wait()
            pltpu.make_async_copy(w_hbm.at[pl.ds(0, EB)], wv, sem).wait()

        def drain_rows(tab, rows, sem):
            pltpu.make_async_copy(tab.at[pl.ds(0, EB)], rows, sem).wait()

        def scale_rows(rows, wv):
            def scale(g, _):
                w16 = wv[pl.ds(g * L, L)]
                for j in range(L):
                    i = g * L + j
                    rows[i, :] = rows[i, :] * w16[j]
                return 0

            lax.fori_loop(0, EB // L, scale, 0)

        for k in range(NCHUNK):
            tab = xtab.at[pl.ds(k * N, N)]

            def zero_slab(i, _):
                pltpu.sync_copy(zbuf, acc.at[pl.ds(my_row0 + i * ZROWS, ZROWS)])
                return 0

            lax.fori_loop(0, ROWS_PER_TILE // ZROWS, zero_slab, 0)
            plsc.subcore_barrier()

            issue_idx(base_edge, srcv0, dstv0, wv0, isem0)
            issue_idx(base_edge + EB, srcv1, dstv1, wv1, isem1)
            drain_idx(srcv0, dstv0, wv0, isem0)
            pltpu.async_copy(tab.at[srcv0], rows0, gsem0)

            def pair(b2, _):
                off = base_edge + 2 * b2 * EB

                drain_idx(srcv1, dstv1, wv1, isem1)

                @pl.when(b2 > 0)
                def _():
                    drain_rows(tab, rows1, ssem1)

                pltpu.async_copy(tab.at[srcv1], rows1, gsem1)

                drain_rows(tab, rows0, gsem0)
                scale_rows(rows0, wv0)
                pltpu.async_copy(rows0, acc.at[dstv0], ssem0, add=True)

                @pl.when(2 * b2 + 2 < NBLK)
                def _():
                    issue_idx(off + 2 * EB, srcv0, dstv0, wv0, isem0)

                @pl.when(2 * b2 + 2 < NBLK)
                def _():
                    drain_idx(srcv0, dstv0, wv0, isem0)
                    drain_rows(tab, rows0, ssem0)
                    pltpu.async_copy(tab.at[srcv0], rows0, gsem0)

                drain_rows(tab, rows1, gsem1)
                scale_rows(rows1, wv1)
                pltpu.async_copy(rows1, acc.at[dstv1], ssem1, add=True)

                @pl.when(2 * b2 + 3 < NBLK)
                def _():
                    issue_idx(off + 3 * EB, srcv1, dstv1, wv1, isem1)

                return 0

            lax.fori_loop(0, NPAIR, pair, 0)
            drain_rows(tab, rows0, gsem0)
            scale_rows(rows0, wv0)
            pltpu.async_copy(rows0, acc.at[dstv0], ssem0, add=True)
            drain_rows(tab, rows0, ssem0)
            drain_rows(tab, rows1, ssem1)
            plsc.subcore_barrier()

            out_row0 = (c * NCHUNK + k) * N_PAD + my_row0
            pltpu.sync_copy(acc.at[pl.ds(my_row0, ROWS_PER_TILE)],
                            out_hbm.at[pl.ds(out_row0, ROWS_PER_TILE)])
            plsc.subcore_barrier()

    return body(xtab, src, dst, w)


def _epilogue(xp, sp_flat, wall, b6, pw, wlin, blin):
    R = 1792
    nbk = N_PAD // R

    def body(x_ref, s00_ref, s01_ref, s02_ref, s10_ref, s11_ref, s12_ref,
             wall_ref, b6_ref, pw_ref, wlin_ref, blin_ref,
             a0_ref, a1_ref, a2_ref, o_ref):
        s0 = s00_ref[...] + s10_ref[...]
        s1 = s01_ref[...] + s11_ref[...]
        s2 = s02_ref[...] + s12_ref[...]
        xs = jnp.concatenate([x_ref[...], s0, s1, s2], axis=1)
        u = (jnp.dot(xs, wall_ref[...], preferred_element_type=jnp.float32)
             + b6_ref[...])
        z = jax.nn.sigmoid(u[:, 0:3])
        ht = jnp.tanh(u[:, 3:6])
        hn = (1.0 - z) * ht
        p = jnp.maximum(hn, 0.0) + pw_ref[0, 0] * jnp.minimum(hn, 0.0)
        o_ref[...] = (jnp.sum(p * wlin_ref[...], axis=1, keepdims=True)
                      + blin_ref[0, 0])
        a38 = jnp.concatenate([s0, s1, s2[:, :F_IN + 3 - 2 * L]], axis=1)
        a0_ref[...] = a38
        a1_ref[...] = a38
        a2_ref[...] = a38

    def sp_spec(c, k):
        base = (c * NCHUNK + k) * nbk
        return pl.BlockSpec((R, L), lambda i, b=base: (b + i, 0))

    full = lambda shape: pl.BlockSpec(shape, lambda i: (0,) * len(shape))
    return pl.pallas_call(
        body,
        grid=(nbk,),
        in_specs=[
            pl.BlockSpec((R, F_PAD), lambda i: (i, 0)),
            sp_spec(0, 0), sp_spec(0, 1), sp_spec(0, 2),
            sp_spec(1, 0), sp_spec(1, 1), sp_spec(1, 2),
            full((2 * F_PAD, 6)), full((1, 6)), full((1, 1)),
            full((1, 3)), full((1, 1)),
        ],
        out_specs=[
            pl.BlockSpec((R, F_IN + 3), lambda i: (i, 0)),
            pl.BlockSpec((R, F_IN + 3), lambda i: (i, 0)),
            pl.BlockSpec((R, F_IN + 3), lambda i: (i, 0)),
            pl.BlockSpec((R, 1), lambda i: (i, 0)),
        ],
        out_shape=[
            jax.ShapeDtypeStruct((N, F_IN + 3), jnp.float32),
            jax.ShapeDtypeStruct((N, F_IN + 3), jnp.float32),
            jax.ShapeDtypeStruct((N, F_IN + 3), jnp.float32),
            jax.ShapeDtypeStruct((N, 1), jnp.float32),
        ],
    )(xp, sp_flat, sp_flat, sp_flat, sp_flat, sp_flat, sp_flat,
      wall, b6, pw, wlin, blin)


def kernel(x, edge_index, edge_weight, W0_r, W1_r, b_r, W0_z, W1_z, b_z,
           W0_h, W1_h, b_h, prelu_w, W_lin, b_lin):
    xp = jnp.pad(x, ((0, 0), (0, F_PAD - F_IN)))
    xflat = xp.reshape(N, NCHUNK, L).transpose(1, 0, 2).reshape(-1)
    xtab = _sc_format_table(xflat)
    sp_flat = _sc_segment_sum(xtab, edge_index[0], edge_index[1],
                              edge_weight)

    pad_w = lambda m: jnp.pad(m[:F_IN, :], ((0, F_PAD - F_IN), (0, 0)))
    wall = jnp.concatenate([
        jnp.concatenate([pad_w(W0_z), pad_w(W0_h)], axis=1),
        jnp.concatenate([pad_w(W1_z), pad_w(W1_h)], axis=1),
    ], axis=0)
    b6 = jnp.concatenate([b_z, b_h]).reshape(1, 6)
    pw = prelu_w.reshape(1, 1)
    wlin = W_lin.reshape(1, 3)
    blin = b_lin.reshape(1, 1)

    a0, a1, a2, out = _epilogue(xp, sp_flat, wall, b6, pw, wlin, blin)
    return (out, a0, a1, a2)


# speedup vs baseline: 16.2871x; 1.0955x over previous
import functools

import jax
import jax.numpy as jnp
from jax import lax
from jax.experimental import pallas as pl
from jax.experimental.pallas import tpu as pltpu
from jax.experimental.pallas import tpu_sc as plsc

N = 100000
E = 1600000
F_IN = 35
F_PAD = 48
L = 16
NCHUNK = 3
NC = 2
NS = 16
NW = NC * NS
EB = 400
EPT = E // NW
NBLK = EPT // EB
NPAIR = NBLK // 2
N_PAD = 100352
ROWS_PER_TILE = N_PAD // NS
ZROWS = 392


FMT_W = 625


def _sc_format_table(xpflat):
    mesh = plsc.VectorSubcoreMesh(core_axis_name="c", subcore_axis_name="s")
    npw = N // NW

    @functools.partial(
        pl.kernel,
        out_type=jax.ShapeDtypeStruct((NCHUNK * N, L), jnp.float32),
        mesh=mesh,
        compiler_params=pltpu.CompilerParams(use_tc_tiling_on_sc=False),
        scratch_types=[
            pltpu.VMEM((FMT_W * F_PAD,), jnp.float32),
            pltpu.VMEM((FMT_W * F_PAD,), jnp.float32),
            pltpu.VMEM((FMT_W, L), jnp.float32),
            pltpu.VMEM((FMT_W, L), jnp.float32),
            pltpu.VMEM((FMT_W, L), jnp.float32),
            pltpu.SemaphoreType.DMA,
            pltpu.SemaphoreType.DMA,
        ],
    )
    def body(x_hbm, tab_hbm, bin0, bin1, b0, b1, b2, sem0, sem1):
        wid = lax.axis_index("c") * NS + lax.axis_index("s")
        n0w = wid * npw
        bins = (bin0, bin1)
        sems = (sem0, sem1)
        n_it = npw // FMT_W
        bks = (b0, b1, b2)

        def issue(i):
            n0 = n0w + i * FMT_W
            pltpu.async_copy(x_hbm.at[pl.ds(n0 * F_PAD, FMT_W * F_PAD)],
                             bins[i % 2], sems[i % 2])

        issue(0)
        issue(1)
        for i in range(n_it):
            pltpu.make_async_copy(x_hbm.at[pl.ds(0, FMT_W * F_PAD)],
                                  bins[i % 2], sems[i % 2]).wait()
            bi = bins[i % 2]

            def repack(r, _):
                for k in range(NCHUNK):
                    bks[k][r, :] = bi[pl.ds(r * F_PAD + k * L, L)]
                return 0

            lax.fori_loop(0, FMT_W, repack, 0)
            n0 = n0w + i * FMT_W
            for k in range(NCHUNK):
                pltpu.sync_copy(bks[k], tab_hbm.at[pl.ds(k * N + n0, FMT_W)])
            if i + 2 < n_it:
                issue(i + 2)

    return body(xpflat)


def _sc_segment_sum(xtab, src, dst, w):
    mesh = plsc.VectorSubcoreMesh(core_axis_name="c", subcore_axis_name="s")

    @functools.partial(
        pl.kernel,
        out_type=jax.ShapeDtypeStruct((NC * NCHUNK * N_PAD, L), jnp.float32),
        mesh=mesh,
        compiler_params=pltpu.CompilerParams(use_tc_tiling_on_sc=False),
        scratch_types=[
            pltpu.VMEM_SHARED((N_PAD, L), jnp.float32),
            pltpu.VMEM((EB,), jnp.int32),
            pltpu.VMEM((EB,), jnp.int32),
            pltpu.VMEM((EB,), jnp.float32),
            pltpu.VMEM((EB, L), jnp.float32),
            pltpu.VMEM((EB,), jnp.int32),
            pltpu.VMEM((EB,), jnp.int32),
            pltpu.VMEM((EB,), jnp.float32),
            pltpu.VMEM((EB, L), jnp.float32),
            pltpu.VMEM((ZROWS, L), jnp.float32),
            pltpu.SemaphoreType.DMA,
            pltpu.SemaphoreType.DMA,
            pltpu.SemaphoreType.DMA,
            pltpu.SemaphoreType.DMA,
            pltpu.SemaphoreType.DMA,
            pltpu.SemaphoreType.DMA,
        ],
    )
    def body(xtab, src_hbm, dst_hbm, w_hbm, out_hbm,
             acc, srcv0, dstv0, wv0, rows0, srcv1, dstv1, wv1, rows1, zbuf,
             isem0, isem1, gsem0, gsem1, ssem0, ssem1):
        c = lax.axis_index("c")
        s = lax.axis_index("s")
        base_edge = (c * NS + s) * EPT
        my_row0 = s * ROWS_PER_TILE

        def fill_zero(i, _):
            zbuf[i, :] = jnp.zeros((L,), jnp.float32)
            return 0

        lax.fori_loop(0, ZROWS, fill_zero, 0)

        def issue_idx(off, srcv, dstv, wv, sem):
            pltpu.async_copy(src_hbm.at[pl.ds(off, EB)], srcv, sem)
            pltpu.async_copy(dst_hbm.at[pl.ds(off, EB)], dstv, sem)
            pltpu.async_copy(w_hbm.at[pl.ds(off, EB)], wv, sem)

        def drain_idx(srcv, dstv, wv, sem):
            pltpu.make_async_copy(src_hbm.at[pl.ds(0, EB)], srcv, sem).wait()
            pltpu.make_async_copy(dst_hbm.at[pl.ds(0, EB)], dstv, sem).wait()
            pltpu.make_async_copy(w_hbm.at[pl.ds(0, EB)], wv, sem).wait()

        def drain_rows(tab, rows, sem):
            pltpu.make_async_copy(tab.at[pl.ds(0, EB)], rows, sem).wait()

        def scale_rows(rows, wv):
            def scale(g, _):
                w16 = wv[pl.ds(g * L, L)]
                for j in range(L):
                    i = g * L + j
                    rows[i, :] = rows[i, :] * w16[j]
                return 0

            lax.fori_loop(0, EB // L, scale, 0)

        for k in range(NCHUNK):
            tab = xtab.at[pl.ds(k * N, N)]

            def zero_slab(i, _):
                pltpu.sync_copy(zbuf, acc.at[pl.ds(my_row0 + i * ZROWS, ZROWS)])
                return 0

            lax.fori_loop(0, ROWS_PER_TILE // ZROWS, zero_slab, 0)
            plsc.subcore_barrier()

            issue_idx(base_edge, srcv0, dstv0, wv0, isem0)
            issue_idx(base_edge + EB, srcv1, dstv1, wv1, isem1)
            drain_idx(srcv0, dstv0, wv0, isem0)
            pltpu.async_copy(tab.at[srcv0], rows0, gsem0)

            def pair(b2, _):
                off = base_edge + 2 * b2 * EB

                drain_idx(srcv1, dstv1, wv1, isem1)

                @pl.when(b2 > 0)
                def _():
                    drain_rows(tab, rows1, ssem1)

                pltpu.async_copy(tab.at[srcv1], rows1, gsem1)

                drain_rows(tab, rows0, gsem0)
                scale_rows(rows0, wv0)
                pltpu.async_copy(rows0, acc.at[dstv0], ssem0, add=True)

                @pl.when(2 * b2 + 2 < NBLK)
                def _():
                    issue_idx(off + 2 * EB, srcv0, dstv0, wv0, isem0)

                @pl.when(2 * b2 + 2 < NBLK)
                def _():
                    drain_idx(srcv0, dstv0, wv0, isem0)
                    drain_rows(tab, rows0, ssem0)
                    pltpu.async_copy(tab.at[srcv0], rows0, gsem0)

                drain_rows(tab, rows1, gsem1)
                scale_rows(rows1, wv1)
                pltpu.async_copy(rows1, acc.at[dstv1], ssem1, add=True)

                @pl.when(2 * b2 + 3 < NBLK)
                def _():
                    issue_idx(off + 3 * EB, srcv1, dstv1, wv1, isem1)

                return 0

            lax.fori_loop(0, NPAIR, pair, 0)
            drain_rows(tab, rows0, gsem0)
            scale_rows(rows0, wv0)
            pltpu.async_copy(rows0, acc.at[dstv0], ssem0, add=True)
            drain_rows(tab, rows0, ssem0)
            drain_rows(tab, rows1, ssem1)
            plsc.subcore_barrier()

            out_row0 = (c * NCHUNK + k) * N_PAD + my_row0
            pltpu.sync_copy(acc.at[pl.ds(my_row0, ROWS_PER_TILE)],
                            out_hbm.at[pl.ds(out_row0, ROWS_PER_TILE)])
            plsc.subcore_barrier()

    return body(xtab, src, dst, w)


def _epilogue(xp, sp_flat, wall, b6, pw, wlin, blin):
    R = 1792
    nbk = N_PAD // R

    def body(x_ref, s00_ref, s01_ref, s02_ref, s10_ref, s11_ref, s12_ref,
             wall_ref, b6_ref, pw_ref, wlin_ref, blin_ref,
             a0_ref, a1_ref, a2_ref, o_ref):
        s0 = s00_ref[...] + s10_ref[...]
        s1 = s01_ref[...] + s11_ref[...]
        s2 = s02_ref[...] + s12_ref[...]
        xs = jnp.concatenate([x_ref[...], s0, s1, s2], axis=1)
        u = (jnp.dot(xs, wall_ref[...], preferred_element_type=jnp.float32)
             + b6_ref[...])
        z = jax.nn.sigmoid(u[:, 0:3])
        ht = jnp.tanh(u[:, 3:6])
        hn = (1.0 - z) * ht
        p = jnp.maximum(hn, 0.0) + pw_ref[0, 0] * jnp.minimum(hn, 0.0)
        o_ref[...] = (jnp.sum(p * wlin_ref[...], axis=1, keepdims=True)
                      + blin_ref[0, 0])
        a38 = jnp.concatenate([s0, s1, s2[:, :F_IN + 3 - 2 * L]], axis=1)
        a0_ref[...] = a38
        a1_ref[...] = a38
        a2_ref[...] = a38

    def sp_spec(c, k):
        base = (c * NCHUNK + k) * nbk
        return pl.BlockSpec((R, L), lambda i, b=base: (b + i, 0))

    full = lambda shape: pl.BlockSpec(shape, lambda i: (0,) * len(shape))
    return pl.pallas_call(
        body,
        grid=(nbk,),
        in_specs=[
            pl.BlockSpec((R, F_PAD), lambda i: (i, 0)),
            sp_spec(0, 0), sp_spec(0, 1), sp_spec(0, 2),
            sp_spec(1, 0), sp_spec(1, 1), sp_spec(1, 2),
            full((2 * F_PAD, 6)), full((1, 6)), full((1, 1)),
            full((1, 3)), full((1, 1)),
        ],
        out_specs=[
            pl.BlockSpec((R, F_IN + 3), lambda i: (i, 0)),
            pl.BlockSpec((R, F_IN + 3), lambda i: (i, 0)),
            pl.BlockSpec((R, F_IN + 3), lambda i: (i, 0)),
            pl.BlockSpec((R, 1), lambda i: (i, 0)),
        ],
        out_shape=[
            jax.ShapeDtypeStruct((N, F_IN + 3), jnp.float32),
            jax.ShapeDtypeStruct((N, F_IN + 3), jnp.float32),
            jax.ShapeDtypeStruct((N, F_IN + 3), jnp.float32),
            jax.ShapeDtypeStruct((N, 1), jnp.float32),
        ],
    )(xp, sp_flat, sp_flat, sp_flat, sp_flat, sp_flat, sp_flat,
      wall, b6, pw, wlin, blin)


def kernel(x, edge_index, edge_weight, W0_r, W1_r, b_r, W0_z, W1_z, b_z,
           W0_h, W1_h, b_h, prelu_w, W_lin, b_lin):
    xp = jnp.pad(x, ((0, 0), (0, F_PAD - F_IN)))
    xpflat = jnp.pad(x, ((0, 0), (0, F_PAD - F_IN))).reshape(-1)
    xtab = _sc_format_table(xpflat)
    sp_flat = _sc_segment_sum(xtab, edge_index[0], edge_index[1],
                              edge_weight)

    pad_w = lambda m: jnp.pad(m[:F_IN, :], ((0, F_PAD - F_IN), (0, 0)))
    wall = jnp.concatenate([
        jnp.concatenate([pad_w(W0_z), pad_w(W0_h)], axis=1),
        jnp.concatenate([pad_w(W1_z), pad_w(W1_h)], axis=1),
    ], axis=0)
    b6 = jnp.concatenate([b_z, b_h]).reshape(1, 6)
    pw = prelu_w.reshape(1, 1)
    wlin = W_lin.reshape(1, 3)
    blin = b_lin.reshape(1, 1)

    a0, a1, a2, out = _epilogue(xp, sp_flat, wall, b6, pw, wlin, blin)
    return (out, a0, a1, a2)
